# Initial kernel scaffold; baseline (speedup 1.0000x reference)
import jax
import jax.numpy as jnp
from jax.experimental import pallas as pl


def kernel(boxes, dimensions, classification):
    raise NotImplementedError("write your pallas kernel here")


# speedup vs baseline: 22.3595x; 22.3595x over previous
import jax
import jax.numpy as jnp
import numpy as np
from jax.experimental import pallas as pl
from jax.experimental.pallas import tpu as pltpu

SCORE_THRESHOLD = 0.05
NMS_THRESHOLD = 0.5
MAX_DETECTIONS = 100
N_BOXES = 5000
N_PAD = 5120
ROWS = N_PAD // 128

NEG_INF = float("-inf")


def _nms_body(x1_ref, y1_ref, x2_ref, y2_ref, cls_ref, g_ref,
              out_rows_ref, out_scores_ref, out_oris_ref, out_valid_ref):
    c0 = jnp.maximum(cls_ref[0], cls_ref[4])
    c1 = jnp.maximum(cls_ref[1], cls_ref[5])
    c2 = jnp.maximum(cls_ref[2], cls_ref[6])
    c3 = jnp.maximum(cls_ref[3], cls_ref[7])
    best = c0
    ori = jnp.zeros_like(c0, dtype=jnp.int32)
    for k, c in ((1, c1), (2, c2), (3, c3)):
        upd = c > best
        ori = jnp.where(upd, k, ori)
        best = jnp.maximum(best, c)
    scores = best

    x1 = x1_ref[...]
    y1 = y1_ref[...]
    x2 = x2_ref[...]
    y2 = y2_ref[...]
    areas = (x2 - x1) * (y2 - y1)

    idx2d = jax.lax.broadcasted_iota(jnp.int32, (ROWS, 128), 0) * 128 + \
        jax.lax.broadcasted_iota(jnp.int32, (ROWS, 128), 1)
    lane = jax.lax.broadcasted_iota(jnp.int32, (1, 128), 1)

    avail0 = scores > SCORE_THRESHOLD

    out_rows_ref[...] = jnp.full((128, 16), -1.0, dtype=jnp.float32)

    def body(t, carry):
        avail_i, keep_s, keep_o, count = carry
        avail = avail_i > 0
        any_avail = jnp.any(avail)
        s = jnp.where(avail, scores, NEG_INF)
        m = jnp.max(s)
        cand = jnp.where(s == m, idx2d, N_PAD)
        j = jnp.min(cand)

        selm = idx2d == j
        x1j = jnp.sum(jnp.where(selm, x1, 0.0))
        y1j = jnp.sum(jnp.where(selm, y1, 0.0))
        x2j = jnp.sum(jnp.where(selm, x2, 0.0))
        y2j = jnp.sum(jnp.where(selm, y2, 0.0))
        orij = jnp.sum(jnp.where(selm, ori, 0))
        areaj = (x2j - x1j) * (y2j - y1j)

        xx1 = jnp.maximum(x1j, x1)
        yy1 = jnp.maximum(y1j, y1)
        xx2 = jnp.minimum(x2j, x2)
        yy2 = jnp.minimum(y2j, y2)
        w = jnp.maximum(0.0, xx2 - xx1)
        h = jnp.maximum(0.0, yy2 - yy1)
        inter = w * h
        iou = inter / (areaj + areas - inter + 1e-9)
        new_avail = avail & ~(iou > NMS_THRESHOLD) & ~selm
        avail_i = jnp.where(any_avail, new_avail.astype(jnp.int32), avail_i)

        slotsel = (lane == t) & any_avail
        keep_s = jnp.where(slotsel, m, keep_s)
        keep_o = jnp.where(slotsel, orij, keep_o)

        @pl.when(any_avail)
        def _():
            out_rows_ref[pl.ds(t, 1), :] = g_ref[pl.ds(j, 1), :]

        count = count + any_avail.astype(jnp.int32)
        return avail_i, keep_s, keep_o, count

    init = (avail0.astype(jnp.int32),
            jnp.zeros((1, 128), jnp.float32),
            jnp.zeros((1, 128), jnp.int32),
            jnp.int32(0))
    _, keep_s, keep_o, count = jax.lax.fori_loop(0, MAX_DETECTIONS, body, init)

    valid = lane < count
    out_scores_ref[...] = jnp.where(valid, keep_s, -1.0)
    out_oris_ref[...] = jnp.where(valid, keep_o, -1)
    out_valid_ref[...] = valid.astype(jnp.int32)


def kernel(boxes, dimensions, classification):
    f32 = jnp.float32
    b4 = jnp.pad(boxes[:, :4], ((0, N_PAD - N_BOXES), (0, 0)))
    x1 = b4[:, 0].reshape(ROWS, 128)
    y1 = b4[:, 1].reshape(ROWS, 128)
    x2 = b4[:, 2].reshape(ROWS, 128)
    y2 = b4[:, 3].reshape(ROWS, 128)
    clsT = jnp.pad(classification, ((0, N_PAD - N_BOXES), (0, 0))).T
    clsT = clsT.reshape(8, ROWS, 128)
    g = jnp.concatenate(
        [boxes, dimensions, jnp.zeros((N_BOXES, 1), f32)], axis=1)
    g = jnp.pad(g, ((0, N_PAD - N_BOXES), (0, 0)))

    out_rows, out_scores, out_oris, out_valid = pl.pallas_call(
        _nms_body,
        out_shape=(
            jax.ShapeDtypeStruct((128, 16), f32),
            jax.ShapeDtypeStruct((1, 128), f32),
            jax.ShapeDtypeStruct((1, 128), jnp.int32),
            jax.ShapeDtypeStruct((1, 128), jnp.int32),
        ),
    )(x1, y1, x2, y2, clsT, g)

    valid = out_valid[0, :MAX_DETECTIONS] > 0
    out_boxes = out_rows[:MAX_DETECTIONS, :12]
    out_dims = out_rows[:MAX_DETECTIONS, 12:15]
    out_s = out_scores[0, :MAX_DETECTIONS]
    out_labels = jnp.where(valid, 0, -1)
    out_o = out_oris[0, :MAX_DETECTIONS]
    return (jnp.asarray(out_boxes, dtype=jnp.float32),
            jnp.asarray(out_dims, dtype=jnp.float32),
            jnp.asarray(out_s, dtype=jnp.float32),
            jnp.asarray(out_labels, dtype=jnp.int64),
            jnp.asarray(out_o, dtype=jnp.int64))
